# baseline (speedup 1.0000x reference)
import functools

import jax
import jax.numpy as jnp
from jax import lax
from jax.experimental import pallas as pl
from jax.experimental.pallas import tpu as pltpu
from jax.experimental.pallas import tpu_sc as plsc

_N = 100000
_C = 3125
_K = 25
_CPAD = 3128
_NW = 32
_L = 16

_CH = 256
_NFULL = _N // _CH
_TAIL_START = _NFULL * _CH
_TAIL = _N - _TAIL_START
_MAXI = -(-_NFULL // _NW)


def _body(vert_hbm, ctrlxy_hbm, ctrlz_hbm, nbr_hbm, dist_hbm, out_hbm,
          ctrlxy_v, ctrlz_v, nbr_v0, dist_v0, vert_v0, out_v0,
          nbr_v1, dist_v1, vert_v1, out_v1,
          nbr_t, dist_t, vert_t, out_t,
          sem_in0, sem_in1, sem_out0, sem_out1):
    wid = lax.axis_index("s") * 2 + lax.axis_index("c")

    nbr_b = [nbr_v0, nbr_v1]
    dist_b = [dist_v0, dist_v1]
    vert_b = [vert_v0, vert_v1]
    out_b = [out_v0, out_v1]
    sem_in = [sem_in0, sem_in1]
    sem_out = [sem_out0, sem_out1]

    pltpu.sync_copy(ctrlxy_hbm, ctrlxy_v)
    pltpu.sync_copy(ctrlz_hbm, ctrlz_v)

    hi_mask = jnp.full((_L,), -65536, jnp.int32)

    def in_descs(c, bs):
        s0 = pl.multiple_of(c * _CH, _CH)
        return (
            pltpu.make_async_copy(nbr_hbm.at[:, pl.ds(s0, _CH)],
                                  nbr_b[bs], sem_in[bs]),
            pltpu.make_async_copy(dist_hbm.at[:, pl.ds(s0, _CH)],
                                  dist_b[bs], sem_in[bs]),
            pltpu.make_async_copy(vert_hbm.at[:, pl.ds(s0, _CH)],
                                  vert_b[bs], sem_in[bs]),
        )

    def out_desc(c, bs):
        s0 = pl.multiple_of(c * _CH, _CH)
        return pltpu.make_async_copy(out_b[bs],
                                     out_hbm.at[:, pl.ds(s0, _CH)],
                                     sem_out[bs])

    def compute(nv, dv, vv, ov, nblocks):
        def one_block(v0):
            z = jnp.zeros((_L,), jnp.float32)
            axs, ays, azs, wss = [z] * 4, [z] * 4, [z] * 4, [z] * 4
            for j in range(_K):
                p = j % 4
                nb = nv[j, pl.ds(v0, _L)]
                dj = dv[j, pl.ds(v0, _L)]
                w = jnp.exp(dj * (-4.5))
                wss[p] = wss[p] + w
                g = plsc.load_gather(ctrlxy_v, [nb])
                cx = plsc.bitcast(g << 16, jnp.float32)
                cy = plsc.bitcast(g & hi_mask, jnp.float32)
                cz = plsc.load_gather(ctrlz_v, [nb])
                axs[p] = axs[p] + w * cx
                ays[p] = ays[p] + w * cy
                azs[p] = azs[p] + w * cz
            ax = (axs[0] + axs[1]) + (axs[2] + axs[3])
            ay = (ays[0] + ays[1]) + (ays[2] + ays[3])
            az = (azs[0] + azs[1]) + (azs[2] + azs[3])
            ws = (wss[0] + wss[1]) + (wss[2] + wss[3])
            inv = 1.0 / jnp.maximum(ws, 0.01)
            ov[0, pl.ds(v0, _L)] = vv[0, pl.ds(v0, _L)] + ax * inv
            ov[1, pl.ds(v0, _L)] = vv[1, pl.ds(v0, _L)] + ay * inv
            ov[2, pl.ds(v0, _L)] = vv[2, pl.ds(v0, _L)] + az * inv

        def block(b, carry):
            one_block(b * (2 * _L))
            one_block(b * (2 * _L) + _L)
            return carry

        lax.fori_loop(0, nblocks // 2, block, 0)

    for i in range(_MAXI + 1):
        if i < _MAXI:
            c = wid + _NW * i

            @pl.when(c < _NFULL)
            def _(c=c, bs=i % 2):
                for d in in_descs(c, bs):
                    d.start()

        if i > 0:
            c = wid + _NW * (i - 1)
            bs = (i - 1) % 2
            if i - 1 >= 2:
                c2 = wid + _NW * (i - 3)

                @pl.when(c2 < _NFULL)
                def _(c2=c2, bs=bs):
                    out_desc(c2, bs).wait()

            @pl.when(c < _NFULL)
            def _(c=c, bs=bs):
                for d in in_descs(c, bs):
                    d.wait()
                compute(nbr_b[bs], dist_b[bs], vert_b[bs], out_b[bs],
                        _CH // _L)
                out_desc(c, bs).start()

    for k in (_MAXI - 2, _MAXI - 1):
        c = wid + _NW * k

        @pl.when(c < _NFULL)
        def _(c=c, bs=k % 2):
            out_desc(c, bs).wait()

    @pl.when(wid == _NW - 1)
    def _():
        pltpu.sync_copy(nbr_hbm.at[:, pl.ds(_TAIL_START, _TAIL)], nbr_t)
        pltpu.sync_copy(dist_hbm.at[:, pl.ds(_TAIL_START, _TAIL)], dist_t)
        pltpu.sync_copy(vert_hbm.at[:, pl.ds(_TAIL_START, _TAIL)], vert_t)
        compute(nbr_t, dist_t, vert_t, out_t, _TAIL // _L)
        pltpu.sync_copy(out_t, out_hbm.at[:, pl.ds(_TAIL_START, _TAIL)])


_mesh = plsc.VectorSubcoreMesh(core_axis_name="c", subcore_axis_name="s")

_sc_call = functools.partial(
    pl.kernel,
    mesh=_mesh,
    compiler_params=pltpu.CompilerParams(needs_layout_passes=False),
    out_type=jax.ShapeDtypeStruct((3, _N), jnp.float32),
    scratch_types=[
        pltpu.VMEM((_CPAD,), jnp.int32),
        pltpu.VMEM((_CPAD,), jnp.float32),
        pltpu.VMEM((_K, _CH), jnp.int32),
        pltpu.VMEM((_K, _CH), jnp.float32),
        pltpu.VMEM((3, _CH), jnp.float32),
        pltpu.VMEM((3, _CH), jnp.float32),
        pltpu.VMEM((_K, _CH), jnp.int32),
        pltpu.VMEM((_K, _CH), jnp.float32),
        pltpu.VMEM((3, _CH), jnp.float32),
        pltpu.VMEM((3, _CH), jnp.float32),
        pltpu.VMEM((_K, _TAIL), jnp.int32),
        pltpu.VMEM((_K, _TAIL), jnp.float32),
        pltpu.VMEM((3, _TAIL), jnp.float32),
        pltpu.VMEM((3, _TAIL), jnp.float32),
        pltpu.SemaphoreType.DMA,
        pltpu.SemaphoreType.DMA,
        pltpu.SemaphoreType.DMA,
        pltpu.SemaphoreType.DMA,
    ],
)(_body)


def kernel(vertices, control_def, neighbours, neighbour_dists):
    xb = jax.lax.bitcast_convert_type(
        control_def[:, 0].astype(jnp.bfloat16), jnp.uint16).astype(jnp.uint32)
    yb = jax.lax.bitcast_convert_type(
        control_def[:, 1].astype(jnp.bfloat16), jnp.uint16).astype(jnp.uint32)
    ctrl_xy = jnp.pad((xb | (yb << 16)).astype(jnp.int32), (0, _CPAD - _C))
    ctrl_z = jnp.pad(control_def[:, 2], (0, _CPAD - _C))
    out_t = _sc_call(vertices.T, ctrl_xy, ctrl_z,
                     neighbours.astype(jnp.int32).T, neighbour_dists.T)
    return out_t.T


# speedup vs baseline: 1.0689x; 1.0689x over previous
import functools

import jax
import jax.numpy as jnp
from jax import lax
from jax.experimental import pallas as pl
from jax.experimental.pallas import tpu as pltpu
from jax.experimental.pallas import tpu_sc as plsc

_N = 100000
_C = 3125
_K = 25
_CPAD = 3128
_NW = 32
_L = 16

_CH = 256
_NFULL = _N // _CH
_TAIL_START = _NFULL * _CH
_TAIL = _N - _TAIL_START
_MAXI = -(-_NFULL // _NW)


def _body(vert_hbm, ctrlxy_hbm, ctrlz_hbm, nbr_hbm, dist_hbm, out_hbm,
          ctrlxy_v, ctrlz_v, nbr_v0, dist_v0, vert_v0, out_v0,
          nbr_v1, dist_v1, vert_v1, out_v1,
          nbr_t, dist_t, vert_t, out_t,
          sem_in0, sem_in1, sem_out0, sem_out1):
    wid = lax.axis_index("s") * 2 + lax.axis_index("c")

    nbr_b = [nbr_v0, nbr_v1]
    dist_b = [dist_v0, dist_v1]
    vert_b = [vert_v0, vert_v1]
    out_b = [out_v0, out_v1]
    sem_in = [sem_in0, sem_in1]
    sem_out = [sem_out0, sem_out1]

    pltpu.sync_copy(ctrlxy_hbm, ctrlxy_v)
    pltpu.sync_copy(ctrlz_hbm, ctrlz_v)

    hi_mask = jnp.full((_L,), -65536, jnp.int32)

    def in_descs(c, bs):
        s0 = pl.multiple_of(c * _CH, _CH)
        return (
            pltpu.make_async_copy(nbr_hbm.at[:, pl.ds(s0, _CH)],
                                  nbr_b[bs], sem_in[bs]),
            pltpu.make_async_copy(dist_hbm.at[:, pl.ds(s0, _CH)],
                                  dist_b[bs], sem_in[bs]),
            pltpu.make_async_copy(vert_hbm.at[:, pl.ds(s0, _CH)],
                                  vert_b[bs], sem_in[bs]),
        )

    def out_desc(c, bs):
        s0 = pl.multiple_of(c * _CH, _CH)
        return pltpu.make_async_copy(out_b[bs],
                                     out_hbm.at[:, pl.ds(s0, _CH)],
                                     sem_out[bs])

    def compute(nv, dv, vv, ov, nblocks):
        def block(b, carry):
            v0 = b * _L
            z = jnp.zeros((_L,), jnp.float32)
            axs, ays, azs, wss = [z] * 4, [z] * 4, [z] * 4, [z] * 4
            for j in range(_K):
                p = j % 4
                nb = nv[j, pl.ds(v0, _L)]
                dj = dv[j, pl.ds(v0, _L)]
                w = jnp.exp(dj * (-4.5))
                wss[p] = wss[p] + w
                g = plsc.load_gather(ctrlxy_v, [nb])
                cx = plsc.bitcast(g << 16, jnp.float32)
                cy = plsc.bitcast(g & hi_mask, jnp.float32)
                cz = plsc.load_gather(ctrlz_v, [nb])
                axs[p] = axs[p] + w * cx
                ays[p] = ays[p] + w * cy
                azs[p] = azs[p] + w * cz
            ax = (axs[0] + axs[1]) + (axs[2] + axs[3])
            ay = (ays[0] + ays[1]) + (ays[2] + ays[3])
            az = (azs[0] + azs[1]) + (azs[2] + azs[3])
            ws = (wss[0] + wss[1]) + (wss[2] + wss[3])
            inv = 1.0 / jnp.maximum(ws, 0.01)
            ov[0, pl.ds(v0, _L)] = vv[0, pl.ds(v0, _L)] + ax * inv
            ov[1, pl.ds(v0, _L)] = vv[1, pl.ds(v0, _L)] + ay * inv
            ov[2, pl.ds(v0, _L)] = vv[2, pl.ds(v0, _L)] + az * inv
            return carry

        lax.fori_loop(0, nblocks, block, 0)

    def guarded(r, fn):
        if _NW - 1 + _NW * r < _NFULL:
            fn()
        else:
            pl.when(wid + _NW * r < _NFULL)(fn)

    for i in range(_MAXI + 1):
        if i < _MAXI:

            def start_in(i=i, bs=i % 2):
                for d in in_descs(wid + _NW * i, bs):
                    d.start()

            guarded(i, start_in)

        if i > 0:
            bs = (i - 1) % 2
            if i - 1 >= 2:
                guarded(i - 3,
                        lambda i=i, bs=bs: out_desc(wid + _NW * (i - 3),
                                                    bs).wait())

            def do_compute(i=i, bs=bs):
                c = wid + _NW * (i - 1)
                for d in in_descs(c, bs):
                    d.wait()
                compute(nbr_b[bs], dist_b[bs], vert_b[bs], out_b[bs],
                        _CH // _L)
                out_desc(c, bs).start()

            guarded(i - 1, do_compute)

    for k in (_MAXI - 2, _MAXI - 1):
        guarded(k, lambda k=k, bs=k % 2: out_desc(wid + _NW * k, bs).wait())

    @pl.when(wid == _NW - 1)
    def _():
        pltpu.sync_copy(nbr_hbm.at[:, pl.ds(_TAIL_START, _TAIL)], nbr_t)
        pltpu.sync_copy(dist_hbm.at[:, pl.ds(_TAIL_START, _TAIL)], dist_t)
        pltpu.sync_copy(vert_hbm.at[:, pl.ds(_TAIL_START, _TAIL)], vert_t)
        compute(nbr_t, dist_t, vert_t, out_t, _TAIL // _L)
        pltpu.sync_copy(out_t, out_hbm.at[:, pl.ds(_TAIL_START, _TAIL)])


_mesh = plsc.VectorSubcoreMesh(core_axis_name="c", subcore_axis_name="s")

_sc_call = functools.partial(
    pl.kernel,
    mesh=_mesh,
    compiler_params=pltpu.CompilerParams(needs_layout_passes=False),
    out_type=jax.ShapeDtypeStruct((3, _N), jnp.float32),
    scratch_types=[
        pltpu.VMEM((_CPAD,), jnp.int32),
        pltpu.VMEM((_CPAD,), jnp.float32),
        pltpu.VMEM((_K, _CH), jnp.int32),
        pltpu.VMEM((_K, _CH), jnp.float32),
        pltpu.VMEM((3, _CH), jnp.float32),
        pltpu.VMEM((3, _CH), jnp.float32),
        pltpu.VMEM((_K, _CH), jnp.int32),
        pltpu.VMEM((_K, _CH), jnp.float32),
        pltpu.VMEM((3, _CH), jnp.float32),
        pltpu.VMEM((3, _CH), jnp.float32),
        pltpu.VMEM((_K, _TAIL), jnp.int32),
        pltpu.VMEM((_K, _TAIL), jnp.float32),
        pltpu.VMEM((3, _TAIL), jnp.float32),
        pltpu.VMEM((3, _TAIL), jnp.float32),
        pltpu.SemaphoreType.DMA,
        pltpu.SemaphoreType.DMA,
        pltpu.SemaphoreType.DMA,
        pltpu.SemaphoreType.DMA,
    ],
)(_body)


def kernel(vertices, control_def, neighbours, neighbour_dists):
    xb = jax.lax.bitcast_convert_type(
        control_def[:, 0].astype(jnp.bfloat16), jnp.uint16).astype(jnp.uint32)
    yb = jax.lax.bitcast_convert_type(
        control_def[:, 1].astype(jnp.bfloat16), jnp.uint16).astype(jnp.uint32)
    ctrl_xy = jnp.pad((xb | (yb << 16)).astype(jnp.int32), (0, _CPAD - _C))
    ctrl_z = jnp.pad(control_def[:, 2], (0, _CPAD - _C))
    out_t = _sc_call(vertices.T, ctrl_xy, ctrl_z,
                     neighbours.astype(jnp.int32).T, neighbour_dists.T)
    return out_t.T
